# baseline (speedup 1.0000x reference)
import functools

import jax
import jax.numpy as jnp
from jax import lax
from jax.experimental import pallas as pl
from jax.experimental.pallas import tpu as pltpu
from jax.experimental.pallas import tpu_sc as plsc

DIM = 64
NUM_WORKERS = 32
CHUNK = 1024


@functools.cache
def _gather_call(B: int):
    b_per_w = B // NUM_WORKERS
    n_chunks = b_per_w // CHUNK
    mesh = plsc.VectorSubcoreMesh(core_axis_name="c", subcore_axis_name="s")

    @functools.partial(
        pl.kernel,
        mesh=mesh,
        out_type=jax.ShapeDtypeStruct((B, DIM), jnp.float32),
        scratch_types=[
            pltpu.VMEM((CHUNK,), jnp.int32),
            pltpu.VMEM((CHUNK, DIM), jnp.float32),
            pltpu.SemaphoreType.DMA,
        ],
        compiler_params=pltpu.CompilerParams(use_tc_tiling_on_sc=False),
    )
    def k(idx_hbm, table_hbm, out_hbm, idx_v, rows_v, sem):
        wid = lax.axis_index("s") * 2 + lax.axis_index("c")
        base = wid * b_per_w

        def body(i, carry):
            off = base + i * CHUNK
            pltpu.sync_copy(idx_hbm.at[pl.ds(off, CHUNK)], idx_v)
            pltpu.async_copy(table_hbm.at[idx_v], rows_v, sem).wait()
            pltpu.sync_copy(rows_v, out_hbm.at[pl.ds(off, CHUNK)])
            return carry

        lax.fori_loop(0, n_chunks, body, 0)

    return k


def kernel(x, table):
    B = x.shape[0] * x.shape[1]
    idx = x.reshape(B).astype(jnp.int32)
    out = _gather_call(B)(idx, table)
    return out.reshape(x.shape[0], x.shape[1], DIM)


# speedup vs baseline: 1.2929x; 1.2929x over previous
import functools

import jax
import jax.numpy as jnp
from jax import lax
from jax.experimental import pallas as pl
from jax.experimental.pallas import tpu as pltpu
from jax.experimental.pallas import tpu_sc as plsc

N_TOK = 4096
SEQ = 200
DIM = 64
NUM_WORKERS = 32
NBUF = 4
ROWS_PER_W = N_TOK // NUM_WORKERS


@functools.cache
def _gather_call():
    mesh = plsc.VectorSubcoreMesh(core_axis_name="c", subcore_axis_name="s")
    scratch = (
        [pltpu.VMEM((SEQ,), jnp.int32) for _ in range(NBUF)]
        + [pltpu.VMEM((SEQ, DIM), jnp.float32) for _ in range(NBUF)]
        + [pltpu.SemaphoreType.DMA for _ in range(2 * NBUF)]
        + [pltpu.SemaphoreType.DMA]
    )

    @functools.partial(
        pl.kernel,
        mesh=mesh,
        out_type=jax.ShapeDtypeStruct((N_TOK, SEQ, 2 * DIM), jnp.float32),
        scratch_types=scratch,
        compiler_params=pltpu.CompilerParams(use_tc_tiling_on_sc=False),
    )
    def k(idx_hbm, table_hbm, out_hbm, *bufs):
        idx_v = bufs[:NBUF]
        rows_v = bufs[NBUF:2 * NBUF]
        sem_i = bufs[2 * NBUF:3 * NBUF]
        sem_o = bufs[3 * NBUF:4 * NBUF]
        sem_g = bufs[4 * NBUF]

        wid = lax.axis_index("s") * 2 + lax.axis_index("c")
        n0 = wid * ROWS_PER_W

        def icopy(chunk, b):
            pltpu.async_copy(
                idx_hbm.at[pl.ds((n0 + chunk) * SEQ, SEQ)], idx_v[b], sem_i[b])

        for b in range(NBUF):
            icopy(b, b)

        n_outer = ROWS_PER_W // NBUF

        def outer(g, carry):
            for b in range(NBUF):
                i = g * NBUF + b
                pltpu.make_async_copy(
                    idx_hbm.at[pl.ds((n0 + i) * SEQ, SEQ)], idx_v[b], sem_i[b]
                ).wait()

                @pl.when(g > 0)
                def _():
                    pltpu.make_async_copy(
                        rows_v[b],
                        out_hbm.at[n0 + i - NBUF, :, pl.ds(0, DIM)],
                        sem_o[b]).wait()

                pltpu.async_copy(
                    table_hbm.at[idx_v[b]], rows_v[b], sem_g).wait()

                @pl.when(g < n_outer - 1)
                def _():
                    icopy(i + NBUF, b)

                pltpu.async_copy(
                    rows_v[b], out_hbm.at[n0 + i, :, pl.ds(0, DIM)], sem_o[b])
            return carry

        lax.fori_loop(0, n_outer, outer, 0)

        last = ROWS_PER_W - NBUF
        for b in range(NBUF):
            pltpu.make_async_copy(
                rows_v[b], out_hbm.at[n0 + last + b, :, pl.ds(0, DIM)],
                sem_o[b]).wait()

    return k


def kernel(x, table):
    idx = x.reshape(N_TOK * SEQ).astype(jnp.int32)
    padded = _gather_call()(idx, table)
    return padded[:, :, :DIM]


# speedup vs baseline: 1.3524x; 1.0460x over previous
import functools

import jax
import jax.numpy as jnp
from jax import lax
from jax.experimental import pallas as pl
from jax.experimental.pallas import tpu as pltpu
from jax.experimental.pallas import tpu_sc as plsc

N_TOK = 4096
SEQ = 200
DIM = 64
NUM_WORKERS = 32
NBUF = 4
CHUNK = 2
ROWS_PER_W = N_TOK // NUM_WORKERS
CH_PER_W = ROWS_PER_W // CHUNK


@functools.cache
def _gather_call():
    mesh = plsc.VectorSubcoreMesh(core_axis_name="c", subcore_axis_name="s")
    scratch = (
        [pltpu.VMEM((CHUNK * SEQ,), jnp.int32) for _ in range(NBUF)]
        + [pltpu.VMEM((CHUNK * SEQ, DIM), jnp.float32) for _ in range(NBUF)]
        + [pltpu.SemaphoreType.DMA for _ in range(2 * NBUF)]
        + [pltpu.SemaphoreType.DMA]
    )

    @functools.partial(
        pl.kernel,
        mesh=mesh,
        out_type=jax.ShapeDtypeStruct((N_TOK, SEQ, 2 * DIM), jnp.float32),
        scratch_types=scratch,
        compiler_params=pltpu.CompilerParams(use_tc_tiling_on_sc=False),
    )
    def k(idx_hbm, table_hbm, out_hbm, *bufs):
        idx_v = bufs[:NBUF]
        rows_v = bufs[NBUF:2 * NBUF]
        sem_i = bufs[2 * NBUF:3 * NBUF]
        sem_o = bufs[3 * NBUF:4 * NBUF]
        sem_g = bufs[4 * NBUF]

        wid = lax.axis_index("s") * 2 + lax.axis_index("c")
        n0 = wid * ROWS_PER_W

        def icopy(chunk, b):
            pltpu.async_copy(
                idx_hbm.at[pl.ds((n0 + chunk * CHUNK) * SEQ, CHUNK * SEQ)],
                idx_v[b], sem_i[b])

        def store(chunk, b, wait):
            for c in range(CHUNK):
                cp = pltpu.make_async_copy(
                    rows_v[b].at[pl.ds(c * SEQ, SEQ)],
                    out_hbm.at[n0 + chunk * CHUNK + c, :, pl.ds(0, DIM)],
                    sem_o[b])
                if wait:
                    cp.wait()
                else:
                    cp.start()

        for b in range(NBUF):
            icopy(b, b)

        def outer(g, carry):
            for b in range(NBUF):
                i = g * NBUF + b
                pltpu.make_async_copy(
                    idx_hbm.at[pl.ds((n0 + i * CHUNK) * SEQ, CHUNK * SEQ)],
                    idx_v[b], sem_i[b]).wait()

                @pl.when(g > 0)
                def _():
                    store(i - NBUF, b, wait=True)

                pltpu.async_copy(
                    table_hbm.at[idx_v[b]], rows_v[b], sem_g).wait()

                @pl.when(g < CH_PER_W // NBUF - 1)
                def _():
                    icopy(i + NBUF, b)

                store(i, b, wait=False)
            return carry

        lax.fori_loop(0, CH_PER_W // NBUF, outer, 0)

        for b in range(NBUF):
            store(CH_PER_W - NBUF + b, b, wait=True)

    return k


def kernel(x, table):
    idx = x.reshape(N_TOK * SEQ).astype(jnp.int32)
    padded = _gather_call()(idx, table)
    return padded[:, :, :DIM]


# speedup vs baseline: 1.3565x; 1.0030x over previous
import functools

import jax
import jax.numpy as jnp
from jax import lax
from jax.experimental import pallas as pl
from jax.experimental.pallas import tpu as pltpu
from jax.experimental.pallas import tpu_sc as plsc

N_TOK = 4096
SEQ = 200
DIM = 64
NUM_WORKERS = 32
NBUF = 4
CHUNK = 2
ROWS_PER_W = N_TOK // NUM_WORKERS
CH_PER_W = ROWS_PER_W // CHUNK


@functools.cache
def _gather_call():
    mesh = plsc.VectorSubcoreMesh(core_axis_name="c", subcore_axis_name="s")
    scratch = (
        [pltpu.VMEM((CHUNK * SEQ,), jnp.int32) for _ in range(NBUF)]
        + [pltpu.VMEM((CHUNK * SEQ, DIM), jnp.float32) for _ in range(NBUF)]
        + [pltpu.SemaphoreType.DMA for _ in range(3 * NBUF)]
    )

    @functools.partial(
        pl.kernel,
        mesh=mesh,
        out_type=jax.ShapeDtypeStruct((N_TOK, SEQ, 2 * DIM), jnp.float32),
        scratch_types=scratch,
        compiler_params=pltpu.CompilerParams(use_tc_tiling_on_sc=False),
    )
    def k(idx_hbm, table_hbm, out_hbm, *bufs):
        idx_v = bufs[:NBUF]
        rows_v = bufs[NBUF:2 * NBUF]
        sem_i = bufs[2 * NBUF:3 * NBUF]
        sem_o = bufs[3 * NBUF:4 * NBUF]
        sem_g = bufs[4 * NBUF:5 * NBUF]

        wid = lax.axis_index("s") * 2 + lax.axis_index("c")
        n0 = wid * ROWS_PER_W

        def icopy(chunk, b):
            pltpu.async_copy(
                idx_hbm.at[pl.ds((n0 + chunk * CHUNK) * SEQ, CHUNK * SEQ)],
                idx_v[b], sem_i[b])

        def store(chunk, b, wait):
            for c in range(CHUNK):
                cp = pltpu.make_async_copy(
                    rows_v[b].at[pl.ds(c * SEQ, SEQ)],
                    out_hbm.at[n0 + chunk * CHUNK + c, :, pl.ds(0, DIM)],
                    sem_o[b])
                if wait:
                    cp.wait()
                else:
                    cp.start()

        for b in range(NBUF):
            icopy(b, b)

        def gwait(b):
            pltpu.make_async_copy(
                table_hbm.at[idx_v[b]], rows_v[b], sem_g[b]).wait()

        def outer(g, carry):
            for b in range(NBUF):
                i = g * NBUF + b
                bp = (b - 1) % NBUF
                pltpu.make_async_copy(
                    idx_hbm.at[pl.ds((n0 + i * CHUNK) * SEQ, CHUNK * SEQ)],
                    idx_v[b], sem_i[b]).wait()

                @pl.when(g > 0)
                def _():
                    store(i - NBUF, b, wait=True)

                pltpu.async_copy(table_hbm.at[idx_v[b]], rows_v[b], sem_g[b])

                @pl.when(i > 0)
                def _():
                    gwait(bp)

                    @pl.when(i - 1 + NBUF < CH_PER_W)
                    def _():
                        icopy(i - 1 + NBUF, bp)

                    store(i - 1, bp, wait=False)
            return carry

        lax.fori_loop(0, CH_PER_W // NBUF, outer, 0)

        bl = (CH_PER_W - 1) % NBUF
        gwait(bl)
        store(CH_PER_W - 1, bl, wait=False)
        for b in range(NBUF):
            store(CH_PER_W - NBUF + b, b, wait=True)

    return k


def kernel(x, table):
    idx = x.reshape(N_TOK * SEQ).astype(jnp.int32)
    padded = _gather_call()(idx, table)
    return padded[:, :, :DIM]


# speedup vs baseline: 1.8552x; 1.3677x over previous
import functools

import jax
import jax.numpy as jnp
from jax import lax
from jax.experimental import pallas as pl
from jax.experimental.pallas import tpu as pltpu
from jax.experimental.pallas import tpu_sc as plsc
from jax.experimental import layout as jex_layout

N_TOK = 4096
SEQ = 200
DIM = 64
NUM_WORKERS = 32
NBUF = 4
CHUNK = 2
ROWS_PER_W = N_TOK // NUM_WORKERS
CH_PER_W = ROWS_PER_W // CHUNK


@functools.cache
def _gather_call():
    mesh = plsc.VectorSubcoreMesh(core_axis_name="c", subcore_axis_name="s")
    scratch = (
        [pltpu.VMEM((CHUNK * SEQ,), jnp.int32) for _ in range(NBUF)]
        + [pltpu.VMEM((CHUNK * SEQ, DIM), jnp.float32) for _ in range(NBUF)]
        + [pltpu.SemaphoreType.DMA for _ in range(3 * NBUF)]
    )

    @functools.partial(
        pl.kernel,
        mesh=mesh,
        out_type=jax.ShapeDtypeStruct((N_TOK, SEQ, 2 * DIM), jnp.float32),
        scratch_types=scratch,
        compiler_params=pltpu.CompilerParams(use_tc_tiling_on_sc=False),
    )
    def k(idx_hbm, table_hbm, out_hbm, *bufs):
        idx_v = bufs[:NBUF]
        rows_v = bufs[NBUF:2 * NBUF]
        sem_i = bufs[2 * NBUF:3 * NBUF]
        sem_o = bufs[3 * NBUF:4 * NBUF]
        sem_g = bufs[4 * NBUF:5 * NBUF]

        wid = lax.axis_index("s") * 2 + lax.axis_index("c")
        n0 = wid * ROWS_PER_W

        def icopy(chunk, b):
            pltpu.async_copy(
                idx_hbm.at[pl.ds((n0 + chunk * CHUNK) * SEQ, CHUNK * SEQ)],
                idx_v[b], sem_i[b])

        def store(chunk, b, wait):
            for c in range(CHUNK):
                cp = pltpu.make_async_copy(
                    rows_v[b].at[pl.ds(c * SEQ, SEQ)],
                    out_hbm.at[n0 + chunk * CHUNK + c, :, pl.ds(0, DIM)],
                    sem_o[b])
                if wait:
                    cp.wait()
                else:
                    cp.start()

        for b in range(NBUF):
            icopy(b, b)

        def gwait(b):
            pltpu.make_async_copy(
                table_hbm.at[idx_v[b]], rows_v[b], sem_g[b]).wait()

        def outer(g, carry):
            for b in range(NBUF):
                i = g * NBUF + b
                bp = (b - 1) % NBUF
                pltpu.make_async_copy(
                    idx_hbm.at[pl.ds((n0 + i * CHUNK) * SEQ, CHUNK * SEQ)],
                    idx_v[b], sem_i[b]).wait()

                @pl.when(g > 0)
                def _():
                    store(i - NBUF, b, wait=True)

                pltpu.async_copy(table_hbm.at[idx_v[b]], rows_v[b], sem_g[b])

                @pl.when(i > 0)
                def _():
                    gwait(bp)

                    @pl.when(i - 1 + NBUF < CH_PER_W)
                    def _():
                        icopy(i - 1 + NBUF, bp)

                    store(i - 1, bp, wait=False)
            return carry

        lax.fori_loop(0, CH_PER_W // NBUF, outer, 0)

        bl = (CH_PER_W - 1) % NBUF
        gwait(bl)
        store(CH_PER_W - 1, bl, wait=False)
        for b in range(NBUF):
            store(CH_PER_W - NBUF + b, b, wait=True)

    return k


def kernel(x, table):
    idx = x.reshape(N_TOK * SEQ).astype(jnp.int32)
    table_rm = jex_layout.with_layout_constraint(
        table, jex_layout.Layout(major_to_minor=(0, 1), tiling=((8,),)))
    padded = _gather_call()(idx, table_rm)
    return padded[:, :, :DIM]
